# Initial kernel scaffold; baseline (speedup 1.0000x reference)
import jax
import jax.numpy as jnp
from jax.experimental import pallas as pl


def kernel(llr):
    raise NotImplementedError("write your pallas kernel here")


# speedup vs baseline: 49.1738x; 49.1738x over previous
import functools

import jax
import jax.numpy as jnp
import numpy as np
from jax.experimental import pallas as pl
from jax.experimental.pallas import tpu as pltpu

_PCM = np.array([
    [1, 0, 1, 0, 1, 0, 1, 0, 1, 0, 1, 0, 1, 0, 1, 0, 1, 0, 1, 0, 1, 0, 1, 0, 1, 0, 1, 0, 1, 0, 1],
    [0, 1, 1, 0, 0, 1, 1, 0, 0, 1, 1, 0, 0, 1, 1, 0, 0, 1, 1, 0, 0, 1, 1, 0, 0, 1, 1, 0, 0, 1, 1],
    [0, 0, 0, 1, 1, 1, 1, 0, 0, 0, 0, 1, 1, 1, 1, 0, 0, 0, 0, 1, 1, 1, 1, 0, 0, 0, 0, 1, 1, 1, 1],
    [0, 0, 0, 0, 0, 0, 0, 1, 1, 1, 1, 1, 1, 1, 1, 0, 0, 0, 0, 0, 0, 0, 0, 1, 1, 1, 1, 1, 1, 1, 1],
    [0, 0, 0, 0, 0, 0, 0, 0, 0, 0, 0, 0, 0, 0, 0, 1, 1, 1, 1, 1, 1, 1, 1, 1, 1, 1, 1, 1, 1, 1, 1],
], dtype=np.int64)
_ROLLED = np.stack(np.where(_PCM), axis=1)
_NCHK, _NVAR = _PCM.shape
_E = _ROLLED.shape[0]
_DEG = 16
_COLS = _ROLLED[:, 1].reshape(_NCHK, _DEG)
_NUM_ITER = 20


def _bp_block(llr_rows, s):
    h_r = [llr_rows[int(_COLS[c, k])] for c in range(_NCHK) for k in range(_DEG)]

    def msgs_to_stack(msgs):
        return jnp.concatenate(msgs, axis=0)

    def stack_to_msgs(m):
        return [m[e * s:(e + 1) * s] for e in range(_E)]

    def body(_, carry):
        m_stack, _cs = carry
        msg = stack_to_msgs(m_stack)
        h_e = [None] * _E
        cs = [None] * _NVAR
        for c in range(_NCHK):
            grp = msg[c * _DEG:(c + 1) * _DEG]
            pref = [grp[0]]
            for k in range(1, _DEG):
                pref.append(pref[-1] * grp[k])
            suf = [grp[_DEG - 1]]
            for k in range(_DEG - 2, -1, -1):
                suf.append(suf[-1] * grp[k])
            suf = suf[::-1]
            for k in range(_DEG):
                if k == 0:
                    loo = suf[1]
                elif k == _DEG - 1:
                    loo = pref[_DEG - 2]
                else:
                    loo = pref[k - 1] * suf[k + 1]
                e1 = jnp.clip(1.0 + loo, 1e-07, 2.0 - 1e-07)
                e2 = jnp.clip(1.0 - loo, 1e-07, 2.0 - 1e-07)
                he = jnp.log(e1 / e2)
                e = c * _DEG + k
                h_e[e] = he
                v = int(_COLS[c, k])
                cs[v] = he if cs[v] is None else cs[v] + he
        new_msg = [
            jnp.tanh((cs[int(_COLS[c, k])] - h_e[c * _DEG + k]
                      + h_r[c * _DEG + k]) * 0.5)
            for c in range(_NCHK) for k in range(_DEG)
        ]
        return msgs_to_stack(new_msg), jnp.concatenate(cs, axis=0)

    msg0 = [jnp.tanh(h * 0.5) for h in h_r]
    cs0 = jnp.zeros((_NVAR * s, llr_rows[0].shape[1]), jnp.float32)
    _, cs_fin = jax.lax.fori_loop(0, _NUM_ITER, body, (msgs_to_stack(msg0), cs0))
    out = [cs_fin[v * s:(v + 1) * s] + llr_rows[v] for v in range(_NVAR)]
    return jnp.concatenate(out, axis=0)


def _tc_kernel(llr_ref, out_ref, *, s):
    llr_rows = [llr_ref[v * s:(v + 1) * s] for v in range(_NVAR)]
    out_ref[...] = _bp_block(llr_rows, s)


@jax.jit
def kernel(llr):
    B = llr.shape[0]
    S = 8
    W = B // S
    WT = 512
    grid = W // WT
    llr2 = llr.T.reshape(_NVAR * S, W)
    out2 = pl.pallas_call(
        functools.partial(_tc_kernel, s=S),
        grid=(grid,),
        in_specs=[pl.BlockSpec((_NVAR * S, WT), lambda i: (0, i))],
        out_specs=pl.BlockSpec((_NVAR * S, WT), lambda i: (0, i)),
        out_shape=jax.ShapeDtypeStruct((_NVAR * S, W), jnp.float32),
    )(llr2)
    return out2.reshape(_NVAR, B).T
